# baseline (speedup 1.0000x reference)
import functools

import jax
import jax.numpy as jnp
from jax import lax
from jax.experimental import pallas as pl
from jax.experimental.pallas import tpu as pltpu
from jax.experimental.pallas import tpu_sc as plsc

LBC = 0.01
BT = 128
BTR = 256


def _lane_bcast(vec, idx_scalar):
    iv = jnp.zeros((16, 1), jnp.int32) + idx_scalar
    return lax.gather(
        vec, iv,
        lax.GatherDimensionNumbers(offset_dims=(), collapsed_slice_dims=(0,),
                                   start_index_map=(0,)),
        (1,), mode=lax.GatherScatterMode.PROMISE_IN_BOUNDS)


def _router_body(x_ref, gw_ref,
                 eid_ref, w_ref, rank_ref, cnt_ref, aux_ref,
                 base_ref, pacc_ref, *, n_blk, n_tok, n_e, k):
    t = pl.program_id(0)
    x = x_ref[...]
    btr = x.shape[0]
    logits = lax.dot_general(gw_ref[...], x, (((1,), (1,)), ((), ())),
                             preferred_element_type=jnp.float32)
    m = jnp.max(logits, axis=0, keepdims=True)
    ex = jnp.exp(logits - m)
    p = ex / jnp.sum(ex, axis=0, keepdims=True)
    rows = lax.broadcasted_iota(jnp.int32, p.shape, 0)
    big = jnp.int32(n_e)
    m1 = jnp.max(p, axis=0, keepdims=True)
    i1 = jnp.min(jnp.where(p == m1, rows, big), axis=0, keepdims=True)
    mask1 = rows == i1
    p2 = jnp.where(mask1, -jnp.inf, p)
    m2 = jnp.max(p2, axis=0, keepdims=True)
    i2 = jnp.min(jnp.where(p2 == m2, rows, big), axis=0, keepdims=True)
    mask2 = rows == i2
    wsum = m1 + m2 + 1e-9
    eid_ref[...] = jnp.concatenate([i1, i2], axis=0)
    w_ref[...] = jnp.concatenate([m1 / wsum, m2 / wsum], axis=0)

    hit = (mask1 | mask2).astype(jnp.float32)

    @pl.when(t == 0)
    def _init():
        base_ref[...] = jnp.zeros_like(base_ref)
        pacc_ref[...] = jnp.zeros_like(pacc_ref)

    tri = (lax.broadcasted_iota(jnp.int32, (btr, btr), 0)
           > lax.broadcasted_iota(jnp.int32, (btr, btr), 1)).astype(jnp.float32)
    excl = lax.dot_general(tri, hit, (((1,), (1,)), ((), ())),
                           preferred_element_type=jnp.float32)
    rank_ref[...] = (excl + base_ref[...]).astype(jnp.int32)
    ones = jnp.ones((1, btr), jnp.float32)
    tot = lax.dot_general(ones, hit, (((1,), (1,)), ((), ())),
                          preferred_element_type=jnp.float32)
    psum = lax.dot_general(ones, p, (((1,), (1,)), ((), ())),
                           preferred_element_type=jnp.float32)
    base_ref[...] += tot
    pacc_ref[...] += psum

    @pl.when(t == n_blk - 1)
    def _fin():
        c = base_ref[...]
        cnt_ref[...] = jnp.concatenate(
            [c, jnp.zeros((1, 16 - c.shape[1]), jnp.float32)], axis=1
        ).astype(jnp.int32)
        f_i = c / (n_tok * k)
        p_i = pacc_ref[...] / n_tok
        aux_ref[0, 0] = LBC * n_e * jnp.sum(f_i * p_i)


def _router(x, gate_w):
    t_tok, h = x.shape
    n_e = gate_w.shape[0]
    n_blk = t_tok // BTR
    return pl.pallas_call(
        functools.partial(_router_body, n_blk=n_blk, n_tok=t_tok, n_e=n_e, k=2),
        grid=(n_blk,),
        in_specs=[
            pl.BlockSpec((BTR, h), lambda t: (t, 0)),
            pl.BlockSpec((n_e, h), lambda t: (0, 0)),
        ],
        out_specs=[
            pl.BlockSpec((2, BTR), lambda t: (0, t)),
            pl.BlockSpec((2, BTR), lambda t: (0, t)),
            pl.BlockSpec((BTR, n_e), lambda t: (t, 0)),
            pl.BlockSpec((1, 16), lambda t: (0, 0)),
            pl.BlockSpec(memory_space=pltpu.SMEM),
        ],
        out_shape=[
            jax.ShapeDtypeStruct((2, t_tok), jnp.int32),
            jax.ShapeDtypeStruct((2, t_tok), jnp.float32),
            jax.ShapeDtypeStruct((t_tok, n_e), jnp.int32),
            jax.ShapeDtypeStruct((1, 16), jnp.int32),
            jax.ShapeDtypeStruct((1, 1), jnp.float32),
        ],
        scratch_shapes=[
            pltpu.VMEM((1, n_e), jnp.float32),
            pltpu.VMEM((1, n_e), jnp.float32),
        ],
    )(x, gate_w)


def _dispatch(eid_flat, w_flat, rank_flat, cnt16, *, t_tok, n_e, npad, nt_pad):
    mesh = plsc.VectorSubcoreMesh(core_axis_name="c", subcore_axis_name="s")

    @functools.partial(
        pl.kernel, mesh=mesh,
        compiler_params=pltpu.CompilerParams(needs_layout_passes=False),
        out_type=[
            jax.ShapeDtypeStruct((npad,), jnp.int32),
            jax.ShapeDtypeStruct((npad,), jnp.float32),
            jax.ShapeDtypeStruct((2 * t_tok,), jnp.int32),
            jax.ShapeDtypeStruct((nt_pad,), jnp.int32),
        ],
        scratch_types=[
            pltpu.VMEM((2 * t_tok,), jnp.int32),
            pltpu.VMEM((2 * t_tok,), jnp.float32),
            pltpu.VMEM((npad,), jnp.float32),
            pltpu.VMEM((t_tok * n_e,), jnp.int32),
            pltpu.VMEM((16,), jnp.int32),
            pltpu.VMEM((npad,), jnp.int32),
            pltpu.VMEM((2 * t_tok,), jnp.int32),
            pltpu.VMEM((nt_pad,), jnp.int32),
            pltpu.VMEM((16,), jnp.int32),
        ],
    )
    def k2(eid_hbm, w_hbm, rank_hbm, cnt_hbm, tok_hbm, ws_hbm, pos_hbm, te_hbm,
           eid_v, w_v, ws_v, rank_v, cnt_v, tok_v, pos_v, te_v, off_v):
        cid = lax.axis_index("c")
        sid = lax.axis_index("s")

        @pl.when((cid == 0) & (sid == 0))
        def _work():
            pltpu.sync_copy(eid_hbm, eid_v)
            pltpu.sync_copy(w_hbm, w_v)
            pltpu.sync_copy(rank_hbm, rank_v)
            pltpu.sync_copy(cnt_hbm, cnt_v)
            lane = lax.iota(jnp.int32, 16)
            cv = jnp.where(lane < n_e, cnt_v[...], 0)
            pc = ((cv + BT - 1) // BT) * BT
            oi = plsc.cumsum(pc)
            off_v[...] = oi - pc

            for jc in range(nt_pad // 16):
                jv = (jc * 16 + lane) * BT
                acc = jnp.zeros((16,), jnp.int32)
                for e in range(n_e):
                    oe_bc = _lane_bcast(oi, e)
                    acc += (jv >= oe_bc).astype(jnp.int32)
                te_v[pl.ds(jc * 16, 16)] = jnp.minimum(acc, n_e - 1)
            pltpu.sync_copy(te_v, te_hbm)

            def zinit(i, carry):
                tok_v[pl.ds(i * 16, 16)] = jnp.zeros((16,), jnp.int32)
                ws_v[pl.ds(i * 16, 16)] = jnp.zeros((16,), jnp.float32)
                return carry
            lax.fori_loop(0, npad // 16, zinit, 0)

            def body(c, carry):
                tvec = c * 16 + lane
                for kk in range(2):
                    ev = eid_v[pl.ds(kk * t_tok + c * 16, 16)]
                    wv = w_v[pl.ds(kk * t_tok + c * 16, 16)]
                    rv = plsc.load_gather(rank_v, [tvec * n_e + ev])
                    ov = plsc.load_gather(off_v, [ev])
                    pv = rv + ov
                    plsc.store_scatter(tok_v, [pv], tvec)
                    plsc.store_scatter(ws_v, [pv], wv)
                    pos_v[pl.ds(kk * t_tok + c * 16, 16)] = pv
                return carry
            lax.fori_loop(0, t_tok // 16, body, 0)
            pltpu.sync_copy(tok_v, tok_hbm)
            pltpu.sync_copy(ws_v, ws_hbm)
            pltpu.sync_copy(pos_v, pos_hbm)

    return k2(eid_flat, w_flat, rank_flat, cnt16)


def _gather_rows(x, tok_sorted, *, npad, h):
    mesh = plsc.VectorSubcoreMesh(core_axis_name="c", subcore_axis_name="s")
    rows_per = npad // 32
    chunk = 16
    n_chunks = rows_per // chunk
    nbuf = 3

    @functools.partial(
        pl.kernel, mesh=mesh,
        compiler_params=pltpu.CompilerParams(needs_layout_passes=False),
        out_type=jax.ShapeDtypeStruct((npad, h), jnp.float32),
        scratch_types=[
            pltpu.VMEM((rows_per,), jnp.int32),
            pltpu.VMEM((chunk, h), jnp.float32),
            pltpu.VMEM((chunk, h), jnp.float32),
            pltpu.VMEM((chunk, h), jnp.float32),
            pltpu.SemaphoreType.DMA,
            pltpu.SemaphoreType.DMA,
            pltpu.SemaphoreType.DMA,
            pltpu.SemaphoreType.DMA,
            pltpu.SemaphoreType.DMA,
            pltpu.SemaphoreType.DMA,
        ],
    )
    def k3(x_hbm, tok_hbm, out_hbm, idx_v, b0, b1, b2,
           sg0, sg1, sg2, ss0, ss1, ss2):
        wid = lax.axis_index("s") * 2 + lax.axis_index("c")
        base = wid * rows_per
        pltpu.sync_copy(tok_hbm.at[pl.ds(base, rows_per)], idx_v)
        buf = (b0, b1, b2)
        sg = (sg0, sg1, sg2)
        ss = (ss0, ss1, ss2)

        def issue(c):
            u = c % nbuf
            return pltpu.async_copy(
                x_hbm.at[idx_v.at[pl.ds(c * chunk, chunk)]], buf[u], sg[u])

        pend = {}
        st = {}
        for c in range(min(nbuf - 1, n_chunks)):
            pend[c] = issue(c)
        for c in range(n_chunks):
            u = c % nbuf
            if c + nbuf - 1 < n_chunks:
                if c >= 1:
                    st[c - 1].wait()
                pend[c + nbuf - 1] = issue(c + nbuf - 1)
            pend.pop(c).wait()
            st[c] = pltpu.async_copy(
                buf[u], out_hbm.at[pl.ds(base + c * chunk, chunk)], ss[u])
        for c in range(max(0, n_chunks - nbuf), n_chunks):
            st[c].wait()

    return k3(x, tok_sorted)


def _ffn_body(te_ref, xs_ref, wg_ref, wu_ref, wd_ref, ws_ref, y_ref):
    x = xs_ref[...].astype(jnp.bfloat16)
    wg = wg_ref[0].astype(jnp.bfloat16)
    wu = wu_ref[0].astype(jnp.bfloat16)
    wd = wd_ref[0].astype(jnp.bfloat16)
    g = lax.dot_general(x, wg, (((1,), (0,)), ((), ())),
                        preferred_element_type=jnp.float32)
    u = lax.dot_general(x, wu, (((1,), (0,)), ((), ())),
                        preferred_element_type=jnp.float32)
    act = ((g / (1.0 + jnp.exp(-g))) * u).astype(jnp.bfloat16)
    y = lax.dot_general(act, wd, (((1,), (0,)), ((), ())),
                        preferred_element_type=jnp.float32)
    bt = y.shape[0]
    ident = (lax.broadcasted_iota(jnp.int32, (bt, bt), 0)
             == lax.broadcasted_iota(jnp.int32, (bt, bt), 1)).astype(jnp.float32)
    wcol = lax.dot_general(ident, ws_ref[0], (((1,), (1,)), ((), ())),
                           preferred_element_type=jnp.float32)
    y_ref[...] = y * wcol


def _ffn(tile_eid, x_sorted, w_gate, w_up, w_down, w_sorted, *, npad, h, f):
    nt = npad // BT
    grid_spec = pltpu.PrefetchScalarGridSpec(
        num_scalar_prefetch=1,
        grid=(nt,),
        in_specs=[
            pl.BlockSpec((BT, h), lambda j, te: (j, 0)),
            pl.BlockSpec((1, h, f), lambda j, te: (te[j], 0, 0)),
            pl.BlockSpec((1, h, f), lambda j, te: (te[j], 0, 0)),
            pl.BlockSpec((1, f, h), lambda j, te: (te[j], 0, 0)),
            pl.BlockSpec((1, 1, BT), lambda j, te: (j, 0, 0)),
        ],
        out_specs=pl.BlockSpec((BT, h), lambda j, te: (j, 0)),
    )
    return pl.pallas_call(
        _ffn_body,
        grid_spec=grid_spec,
        out_shape=jax.ShapeDtypeStruct((npad, h), jnp.float32),
    )(tile_eid, x_sorted, w_gate, w_up, w_down,
      w_sorted.reshape(nt, 1, BT))


def _combine(pos, y, *, t_tok, h):
    mesh = plsc.VectorSubcoreMesh(core_axis_name="c", subcore_axis_name="s")
    toks_per = t_tok // 32
    tc = 8
    n_chunks = toks_per // tc
    nbuf = 3

    @functools.partial(
        pl.kernel, mesh=mesh,
        compiler_params=pltpu.CompilerParams(needs_layout_passes=False),
        out_type=jax.ShapeDtypeStruct((t_tok, h), jnp.float32),
        scratch_types=[
            pltpu.VMEM((toks_per,), jnp.int32),
            pltpu.VMEM((toks_per,), jnp.int32),
            pltpu.VMEM((tc, h), jnp.float32),
            pltpu.VMEM((tc, h), jnp.float32),
            pltpu.VMEM((tc, h), jnp.float32),
            pltpu.VMEM((tc, h), jnp.float32),
            pltpu.VMEM((tc, h), jnp.float32),
            pltpu.VMEM((tc, h), jnp.float32),
            pltpu.SemaphoreType.DMA,
            pltpu.SemaphoreType.DMA,
            pltpu.SemaphoreType.DMA,
            pltpu.SemaphoreType.DMA,
            pltpu.SemaphoreType.DMA,
            pltpu.SemaphoreType.DMA,
        ],
    )
    def k5(pos_hbm, y_hbm, out_hbm,
           p1_v, p2_v, b10, b11, b12, b20, b21, b22,
           sg0, sg1, sg2, ss0, ss1, ss2):
        wid = lax.axis_index("s") * 2 + lax.axis_index("c")
        base = wid * toks_per
        pltpu.sync_copy(pos_hbm.at[pl.ds(base, toks_per)], p1_v)
        pltpu.sync_copy(pos_hbm.at[pl.ds(t_tok + base, toks_per)], p2_v)
        b1 = (b10, b11, b12)
        b2 = (b20, b21, b22)
        sg = (sg0, sg1, sg2)
        ss = (ss0, ss1, ss2)

        def issue(c):
            u = c % nbuf
            cp1 = pltpu.async_copy(y_hbm.at[p1_v.at[pl.ds(c * tc, tc)]], b1[u], sg[u])
            cp2 = pltpu.async_copy(y_hbm.at[p2_v.at[pl.ds(c * tc, tc)]], b2[u], sg[u])
            return cp1, cp2

        pend = {}
        st = {}
        for c in range(min(nbuf - 1, n_chunks)):
            pend[c] = issue(c)
        for c in range(n_chunks):
            u = c % nbuf
            if c + nbuf - 1 < n_chunks:
                if c >= 1:
                    st[c - 1].wait()
                pend[c + nbuf - 1] = issue(c + nbuf - 1)
            cp1, cp2 = pend.pop(c)
            cp1.wait()
            cp2.wait()
            for i in range(tc):
                def hh(hc, carry, i=i, u=u):
                    sl = pl.ds(hc * 16, 16)
                    b1[u][i, sl] += b2[u][i, sl]
                    return carry
                lax.fori_loop(0, h // 16, hh, 0)
            st[c] = pltpu.async_copy(
                b1[u], out_hbm.at[pl.ds(base + c * tc, tc)], ss[u])
        for c in range(max(0, n_chunks - nbuf), n_chunks):
            st[c].wait()

    return k5(pos, y)


def kernel(hidden_states, gate_w, w_gate, w_up, w_down):
    b, s, h = hidden_states.shape
    n_e, _, f = w_gate.shape
    t_tok = b * s
    npad = 2 * t_tok + n_e * BT
    nt = npad // BT
    nt_pad = ((nt + 15) // 16) * 16
    x = hidden_states.reshape(t_tok, h)

    eid, w, rank, cnt, aux = _router(x, gate_w)
    tok_sorted, w_sorted, pos, tile_eid = _dispatch(
        eid.reshape(-1), w.reshape(-1), rank.reshape(-1), cnt.reshape(-1),
        t_tok=t_tok, n_e=n_e, npad=npad, nt_pad=nt_pad)
    x_sorted = _gather_rows(x, tok_sorted, npad=npad, h=h)
    y = _ffn(tile_eid[:nt], x_sorted, w_gate, w_up, w_down, w_sorted,
             npad=npad, h=h, f=f)
    out = _combine(pos, y, t_tok=t_tok, h=h)
    return out.reshape(b, s, h), aux[0, 0]


# speedup vs baseline: 1.1063x; 1.1063x over previous
import functools

import jax
import jax.numpy as jnp
from jax import lax
from jax.experimental import pallas as pl
from jax.experimental.pallas import tpu as pltpu
from jax.experimental.pallas import tpu_sc as plsc

LBC = 0.01
BT = 128
BTR = 256


def _lane_bcast(vec, idx_scalar):
    iv = jnp.zeros((16, 1), jnp.int32) + idx_scalar
    return lax.gather(
        vec, iv,
        lax.GatherDimensionNumbers(offset_dims=(), collapsed_slice_dims=(0,),
                                   start_index_map=(0,)),
        (1,), mode=lax.GatherScatterMode.PROMISE_IN_BOUNDS)


def _router_body(x_ref, gw_ref,
                 eid_ref, w_ref, rank_ref, cnt_ref, aux_ref,
                 base_ref, pacc_ref, *, n_blk, n_tok, n_e, k):
    t = pl.program_id(0)
    x = x_ref[...]
    btr = x.shape[0]
    logits = lax.dot_general(gw_ref[...], x, (((1,), (1,)), ((), ())),
                             preferred_element_type=jnp.float32)
    m = jnp.max(logits, axis=0, keepdims=True)
    ex = jnp.exp(logits - m)
    p = ex / jnp.sum(ex, axis=0, keepdims=True)
    rows = lax.broadcasted_iota(jnp.int32, p.shape, 0)
    big = jnp.int32(n_e)
    m1 = jnp.max(p, axis=0, keepdims=True)
    i1 = jnp.min(jnp.where(p == m1, rows, big), axis=0, keepdims=True)
    mask1 = rows == i1
    p2 = jnp.where(mask1, -jnp.inf, p)
    m2 = jnp.max(p2, axis=0, keepdims=True)
    i2 = jnp.min(jnp.where(p2 == m2, rows, big), axis=0, keepdims=True)
    mask2 = rows == i2
    wsum = m1 + m2 + 1e-9
    eid_ref[...] = jnp.concatenate([i1, i2], axis=0)
    w_ref[...] = jnp.concatenate([m1 / wsum, m2 / wsum], axis=0)

    hit = (mask1 | mask2).astype(jnp.float32)

    @pl.when(t == 0)
    def _init():
        base_ref[...] = jnp.zeros_like(base_ref)
        pacc_ref[...] = jnp.zeros_like(pacc_ref)

    tri = (lax.broadcasted_iota(jnp.int32, (btr, btr), 0)
           > lax.broadcasted_iota(jnp.int32, (btr, btr), 1)).astype(jnp.float32)
    excl = lax.dot_general(tri, hit, (((1,), (1,)), ((), ())),
                           preferred_element_type=jnp.float32)
    rank_ref[...] = (excl + base_ref[...]).astype(jnp.int32)
    ones = jnp.ones((1, btr), jnp.float32)
    tot = lax.dot_general(ones, hit, (((1,), (1,)), ((), ())),
                          preferred_element_type=jnp.float32)
    psum = lax.dot_general(ones, p, (((1,), (1,)), ((), ())),
                           preferred_element_type=jnp.float32)
    base_ref[...] += tot
    pacc_ref[...] += psum

    @pl.when(t == n_blk - 1)
    def _fin():
        c = base_ref[...]
        cnt_ref[...] = jnp.concatenate(
            [c, jnp.zeros((1, 16 - c.shape[1]), jnp.float32)], axis=1
        ).astype(jnp.int32)
        f_i = c / (n_tok * k)
        p_i = pacc_ref[...] / n_tok
        aux_ref[0, 0] = LBC * n_e * jnp.sum(f_i * p_i)


def _router(x, gate_w):
    t_tok, h = x.shape
    n_e = gate_w.shape[0]
    n_blk = t_tok // BTR
    return pl.pallas_call(
        functools.partial(_router_body, n_blk=n_blk, n_tok=t_tok, n_e=n_e, k=2),
        grid=(n_blk,),
        in_specs=[
            pl.BlockSpec((BTR, h), lambda t: (t, 0)),
            pl.BlockSpec((n_e, h), lambda t: (0, 0)),
        ],
        out_specs=[
            pl.BlockSpec((2, BTR), lambda t: (0, t)),
            pl.BlockSpec((2, BTR), lambda t: (0, t)),
            pl.BlockSpec((BTR, n_e), lambda t: (t, 0)),
            pl.BlockSpec((1, 16), lambda t: (0, 0)),
            pl.BlockSpec(memory_space=pltpu.SMEM),
        ],
        out_shape=[
            jax.ShapeDtypeStruct((2, t_tok), jnp.int32),
            jax.ShapeDtypeStruct((2, t_tok), jnp.float32),
            jax.ShapeDtypeStruct((t_tok, n_e), jnp.int32),
            jax.ShapeDtypeStruct((1, 16), jnp.int32),
            jax.ShapeDtypeStruct((1, 1), jnp.float32),
        ],
        scratch_shapes=[
            pltpu.VMEM((1, n_e), jnp.float32),
            pltpu.VMEM((1, n_e), jnp.float32),
        ],
    )(x, gate_w)


def _dispatch(eid_flat, w_flat, rank_flat, cnt16, *, t_tok, n_e, npad, nt_pad):
    mesh = plsc.VectorSubcoreMesh(core_axis_name="c", subcore_axis_name="s")

    @functools.partial(
        pl.kernel, mesh=mesh,
        compiler_params=pltpu.CompilerParams(needs_layout_passes=False),
        out_type=[
            jax.ShapeDtypeStruct((npad,), jnp.int32),
            jax.ShapeDtypeStruct((npad,), jnp.float32),
            jax.ShapeDtypeStruct((2 * t_tok,), jnp.int32),
            jax.ShapeDtypeStruct((nt_pad,), jnp.int32),
        ],
        scratch_types=[
            pltpu.VMEM((2 * t_tok,), jnp.int32),
            pltpu.VMEM((2 * t_tok,), jnp.float32),
            pltpu.VMEM((npad,), jnp.float32),
            pltpu.VMEM((t_tok * n_e,), jnp.int32),
            pltpu.VMEM((16,), jnp.int32),
            pltpu.VMEM((npad,), jnp.int32),
            pltpu.VMEM((2 * t_tok,), jnp.int32),
            pltpu.VMEM((nt_pad,), jnp.int32),
            pltpu.VMEM((16,), jnp.int32),
        ],
    )
    def k2(eid_hbm, w_hbm, rank_hbm, cnt_hbm, tok_hbm, ws_hbm, pos_hbm, te_hbm,
           eid_v, w_v, ws_v, rank_v, cnt_v, tok_v, pos_v, te_v, off_v):
        cid = lax.axis_index("c")
        sid = lax.axis_index("s")

        @pl.when((cid == 0) & (sid == 0))
        def _work():
            pltpu.sync_copy(eid_hbm, eid_v)
            pltpu.sync_copy(w_hbm, w_v)
            pltpu.sync_copy(rank_hbm, rank_v)
            pltpu.sync_copy(cnt_hbm, cnt_v)
            lane = lax.iota(jnp.int32, 16)
            cv = jnp.where(lane < n_e, cnt_v[...], 0)
            pc = ((cv + BT - 1) // BT) * BT
            oi = plsc.cumsum(pc)
            off_v[...] = oi - pc

            for jc in range(nt_pad // 16):
                jv = (jc * 16 + lane) * BT
                acc = jnp.zeros((16,), jnp.int32)
                for e in range(n_e):
                    oe_bc = _lane_bcast(oi, e)
                    acc += (jv >= oe_bc).astype(jnp.int32)
                te_v[pl.ds(jc * 16, 16)] = jnp.minimum(acc, n_e - 1)
            pltpu.sync_copy(te_v, te_hbm)

            @plsc.parallel_loop(0, npad, step=16, unroll=8)
            def zinit(i):
                tok_v[pl.ds(i, 16)] = jnp.zeros((16,), jnp.int32)
                ws_v[pl.ds(i, 16)] = jnp.zeros((16,), jnp.float32)

            def body(c, carry):
                tvec = c * 16 + lane
                for kk in range(2):
                    ev = eid_v[pl.ds(kk * t_tok + c * 16, 16)]
                    wv = w_v[pl.ds(kk * t_tok + c * 16, 16)]
                    rv = plsc.load_gather(rank_v, [tvec * n_e + ev])
                    ov = plsc.load_gather(off_v, [ev])
                    pv = rv + ov
                    plsc.store_scatter(tok_v, [pv], tvec)
                    plsc.store_scatter(ws_v, [pv], wv)
                    pos_v[pl.ds(kk * t_tok + c * 16, 16)] = pv
                return carry
            lax.fori_loop(0, t_tok // 16, body, 0)
            pltpu.sync_copy(tok_v, tok_hbm)
            pltpu.sync_copy(ws_v, ws_hbm)
            pltpu.sync_copy(pos_v, pos_hbm)

    return k2(eid_flat, w_flat, rank_flat, cnt16)


def _gather_rows(x, tok_sorted, *, npad, h):
    mesh = plsc.VectorSubcoreMesh(core_axis_name="c", subcore_axis_name="s")
    rows_per = npad // 32
    chunk = 16
    n_chunks = rows_per // chunk

    @functools.partial(
        pl.kernel, mesh=mesh,
        compiler_params=pltpu.CompilerParams(needs_layout_passes=False),
        out_type=jax.ShapeDtypeStruct((npad, h), jnp.float32),
        scratch_types=[
            pltpu.VMEM((rows_per,), jnp.int32),
            pltpu.VMEM((chunk, h), jnp.float32),
            pltpu.VMEM((chunk, h), jnp.float32),
            pltpu.SemaphoreType.DMA,
            pltpu.SemaphoreType.DMA,
            pltpu.SemaphoreType.DMA,
            pltpu.SemaphoreType.DMA,
        ],
    )
    def k3(x_hbm, tok_hbm, out_hbm, idx_v, bufa, bufb, sga, sgb, ssa, ssb):
        wid = lax.axis_index("s") * 2 + lax.axis_index("c")
        base = wid * rows_per
        pltpu.sync_copy(tok_hbm.at[pl.ds(base, rows_per)], idx_v)
        buf = (bufa, bufb)
        sg = (sga, sgb)
        ss = (ssa, ssb)

        def issue(c):
            u = c % 2
            return pltpu.async_copy(
                x_hbm.at[idx_v.at[pl.ds(c * chunk, chunk)]], buf[u], sg[u])

        pend = {0: issue(0)}
        st = {}
        for c in range(n_chunks):
            u = c % 2
            if c + 1 < n_chunks:
                if c >= 1:
                    st[c - 1].wait()
                pend[c + 1] = issue(c + 1)
            pend.pop(c).wait()
            st[c] = pltpu.async_copy(
                buf[u], out_hbm.at[pl.ds(base + c * chunk, chunk)], ss[u])
        st[n_chunks - 2].wait()
        st[n_chunks - 1].wait()

    return k3(x, tok_sorted)


def _ffn_body(te_ref, xs_ref, wg_ref, wu_ref, wd_ref, ws_ref, y_ref):
    x = xs_ref[...].astype(jnp.bfloat16)
    wg = wg_ref[0].astype(jnp.bfloat16)
    wu = wu_ref[0].astype(jnp.bfloat16)
    wd = wd_ref[0].astype(jnp.bfloat16)
    g = lax.dot_general(x, wg, (((1,), (0,)), ((), ())),
                        preferred_element_type=jnp.float32)
    u = lax.dot_general(x, wu, (((1,), (0,)), ((), ())),
                        preferred_element_type=jnp.float32)
    act = ((g / (1.0 + jnp.exp(-g))) * u).astype(jnp.bfloat16)
    y = lax.dot_general(act, wd, (((1,), (0,)), ((), ())),
                        preferred_element_type=jnp.float32)
    bt = y.shape[0]
    ident = (lax.broadcasted_iota(jnp.int32, (bt, bt), 0)
             == lax.broadcasted_iota(jnp.int32, (bt, bt), 1)).astype(jnp.float32)
    wcol = lax.dot_general(ident, ws_ref[0], (((1,), (1,)), ((), ())),
                           preferred_element_type=jnp.float32)
    y_ref[...] = y * wcol


def _ffn(tile_eid, x_sorted, w_gate, w_up, w_down, w_sorted, *, npad, h, f):
    nt = npad // BT
    grid_spec = pltpu.PrefetchScalarGridSpec(
        num_scalar_prefetch=1,
        grid=(nt,),
        in_specs=[
            pl.BlockSpec((BT, h), lambda j, te: (j, 0)),
            pl.BlockSpec((1, h, f), lambda j, te: (te[j], 0, 0)),
            pl.BlockSpec((1, h, f), lambda j, te: (te[j], 0, 0)),
            pl.BlockSpec((1, f, h), lambda j, te: (te[j], 0, 0)),
            pl.BlockSpec((1, 1, BT), lambda j, te: (j, 0, 0)),
        ],
        out_specs=pl.BlockSpec((BT, h), lambda j, te: (j, 0)),
    )
    return pl.pallas_call(
        _ffn_body,
        grid_spec=grid_spec,
        out_shape=jax.ShapeDtypeStruct((npad, h), jnp.float32),
    )(tile_eid, x_sorted, w_gate, w_up, w_down,
      w_sorted.reshape(nt, 1, BT))


def _combine(pos, y, *, t_tok, h):
    mesh = plsc.VectorSubcoreMesh(core_axis_name="c", subcore_axis_name="s")
    toks_per = t_tok // 32
    tc = 8
    n_chunks = toks_per // tc

    @functools.partial(
        pl.kernel, mesh=mesh,
        compiler_params=pltpu.CompilerParams(needs_layout_passes=False),
        out_type=jax.ShapeDtypeStruct((t_tok, h), jnp.float32),
        scratch_types=[
            pltpu.VMEM((toks_per,), jnp.int32),
            pltpu.VMEM((toks_per,), jnp.int32),
            pltpu.VMEM((tc, h), jnp.float32),
            pltpu.VMEM((tc, h), jnp.float32),
            pltpu.VMEM((tc, h), jnp.float32),
            pltpu.VMEM((tc, h), jnp.float32),
            pltpu.SemaphoreType.DMA,
            pltpu.SemaphoreType.DMA,
            pltpu.SemaphoreType.DMA,
            pltpu.SemaphoreType.DMA,
        ],
    )
    def k5(pos_hbm, y_hbm, out_hbm,
           p1_v, p2_v, b1a, b1b, b2a, b2b, sga, sgb, ssa, ssb):
        wid = lax.axis_index("s") * 2 + lax.axis_index("c")
        base = wid * toks_per
        pltpu.sync_copy(pos_hbm.at[pl.ds(base, toks_per)], p1_v)
        pltpu.sync_copy(pos_hbm.at[pl.ds(t_tok + base, toks_per)], p2_v)
        b1 = (b1a, b1b)
        b2 = (b2a, b2b)
        sg = (sga, sgb)
        ss = (ssa, ssb)

        def issue(c):
            u = c % 2
            cp1 = pltpu.async_copy(y_hbm.at[p1_v.at[pl.ds(c * tc, tc)]], b1[u], sg[u])
            cp2 = pltpu.async_copy(y_hbm.at[p2_v.at[pl.ds(c * tc, tc)]], b2[u], sg[u])
            return cp1, cp2

        pend = {0: issue(0)}
        st = {}
        for c in range(n_chunks):
            u = c % 2
            if c + 1 < n_chunks:
                if c >= 1:
                    st[c - 1].wait()
                pend[c + 1] = issue(c + 1)
            cp1, cp2 = pend.pop(c)
            cp1.wait()
            cp2.wait()
            for i in range(tc):
                @plsc.parallel_loop(0, h, step=16, unroll=8)
                def hh(hx, i=i, u=u):
                    sl = pl.ds(hx, 16)
                    b1[u][i, sl] += b2[u][i, sl]
            st[c] = pltpu.async_copy(b1[u], out_hbm.at[pl.ds(base + c * tc, tc)], ss[u])
        st[n_chunks - 2].wait()
        st[n_chunks - 1].wait()

    return k5(pos, y)


def kernel(hidden_states, gate_w, w_gate, w_up, w_down):
    b, s, h = hidden_states.shape
    n_e, _, f = w_gate.shape
    t_tok = b * s
    npad = 2 * t_tok + n_e * BT
    nt = npad // BT
    nt_pad = ((nt + 15) // 16) * 16
    x = hidden_states.reshape(t_tok, h)

    eid, w, rank, cnt, aux = _router(x, gate_w)
    tok_sorted, w_sorted, pos, tile_eid = _dispatch(
        eid.reshape(-1), w.reshape(-1), rank.reshape(-1), cnt.reshape(-1),
        t_tok=t_tok, n_e=n_e, npad=npad, nt_pad=nt_pad)
    x_sorted = _gather_rows(x, tok_sorted, npad=npad, h=h)
    y = _ffn(tile_eid[:nt], x_sorted, w_gate, w_up, w_down, w_sorted,
             npad=npad, h=h, f=f)
    out = _combine(pos, y, t_tok=t_tok, h=h)
    return out.reshape(b, s, h), aux[0, 0]


# speedup vs baseline: 1.2248x; 1.1072x over previous
import functools

import jax
import jax.numpy as jnp
from jax import lax
from jax.experimental import pallas as pl
from jax.experimental.pallas import tpu as pltpu
from jax.experimental.pallas import tpu_sc as plsc

LBC = 0.01
BT = 128
BTR = 256


def _lane_bcast(vec, idx_scalar):
    iv = jnp.zeros((16, 1), jnp.int32) + idx_scalar
    return lax.gather(
        vec, iv,
        lax.GatherDimensionNumbers(offset_dims=(), collapsed_slice_dims=(0,),
                                   start_index_map=(0,)),
        (1,), mode=lax.GatherScatterMode.PROMISE_IN_BOUNDS)


def _router_body(x_ref, gw_ref,
                 eid_ref, w_ref, rank_ref, cnt_ref, xp_ref, aux_ref,
                 base_ref, pacc_ref, *, n_blk, n_tok, n_e, k):
    t = pl.program_id(0)
    x = x_ref[...]
    btr = x.shape[0]
    hh = x.shape[1] // 2
    xb = x.astype(jnp.bfloat16)
    au = lax.bitcast_convert_type(xb[:, :hh], jnp.uint16).astype(jnp.int32)
    bu = lax.bitcast_convert_type(xb[:, hh:], jnp.uint16).astype(jnp.int32)
    xp_ref[...] = au | (bu << 16)
    logits = lax.dot_general(gw_ref[...], x, (((1,), (1,)), ((), ())),
                             preferred_element_type=jnp.float32)
    m = jnp.max(logits, axis=0, keepdims=True)
    ex = jnp.exp(logits - m)
    p = ex / jnp.sum(ex, axis=0, keepdims=True)
    rows = lax.broadcasted_iota(jnp.int32, p.shape, 0)
    big = jnp.int32(n_e)
    m1 = jnp.max(p, axis=0, keepdims=True)
    i1 = jnp.min(jnp.where(p == m1, rows, big), axis=0, keepdims=True)
    mask1 = rows == i1
    p2 = jnp.where(mask1, -jnp.inf, p)
    m2 = jnp.max(p2, axis=0, keepdims=True)
    i2 = jnp.min(jnp.where(p2 == m2, rows, big), axis=0, keepdims=True)
    mask2 = rows == i2
    wsum = m1 + m2 + 1e-9
    eid_ref[...] = jnp.concatenate([i1, i2], axis=0)
    w_ref[...] = jnp.concatenate([m1 / wsum, m2 / wsum], axis=0)

    hit = (mask1 | mask2).astype(jnp.float32)

    @pl.when(t == 0)
    def _init():
        base_ref[...] = jnp.zeros_like(base_ref)
        pacc_ref[...] = jnp.zeros_like(pacc_ref)

    tri = (lax.broadcasted_iota(jnp.int32, (btr, btr), 0)
           > lax.broadcasted_iota(jnp.int32, (btr, btr), 1)).astype(jnp.float32)
    excl = lax.dot_general(tri, hit, (((1,), (1,)), ((), ())),
                           preferred_element_type=jnp.float32)
    rank_ref[...] = (excl + base_ref[...]).astype(jnp.int32)
    ones = jnp.ones((1, btr), jnp.float32)
    tot = lax.dot_general(ones, hit, (((1,), (1,)), ((), ())),
                          preferred_element_type=jnp.float32)
    psum = lax.dot_general(ones, p, (((1,), (1,)), ((), ())),
                           preferred_element_type=jnp.float32)
    base_ref[...] += tot
    pacc_ref[...] += psum

    @pl.when(t == n_blk - 1)
    def _fin():
        c = base_ref[...]
        cnt_ref[...] = jnp.concatenate(
            [c, jnp.zeros((1, 16 - c.shape[1]), jnp.float32)], axis=1
        ).astype(jnp.int32)
        f_i = c / (n_tok * k)
        p_i = pacc_ref[...] / n_tok
        aux_ref[0, 0] = LBC * n_e * jnp.sum(f_i * p_i)


def _router(x, gate_w):
    t_tok, h = x.shape
    n_e = gate_w.shape[0]
    n_blk = t_tok // BTR
    return pl.pallas_call(
        functools.partial(_router_body, n_blk=n_blk, n_tok=t_tok, n_e=n_e, k=2),
        grid=(n_blk,),
        in_specs=[
            pl.BlockSpec((BTR, h), lambda t: (t, 0)),
            pl.BlockSpec((n_e, h), lambda t: (0, 0)),
        ],
        out_specs=[
            pl.BlockSpec((2, BTR), lambda t: (0, t)),
            pl.BlockSpec((2, BTR), lambda t: (0, t)),
            pl.BlockSpec((BTR, n_e), lambda t: (t, 0)),
            pl.BlockSpec((1, 16), lambda t: (0, 0)),
            pl.BlockSpec((BTR, h // 2), lambda t: (t, 0)),
            pl.BlockSpec(memory_space=pltpu.SMEM),
        ],
        out_shape=[
            jax.ShapeDtypeStruct((2, t_tok), jnp.int32),
            jax.ShapeDtypeStruct((2, t_tok), jnp.float32),
            jax.ShapeDtypeStruct((t_tok, n_e), jnp.int32),
            jax.ShapeDtypeStruct((1, 16), jnp.int32),
            jax.ShapeDtypeStruct((t_tok, h // 2), jnp.int32),
            jax.ShapeDtypeStruct((1, 1), jnp.float32),
        ],
        scratch_shapes=[
            pltpu.VMEM((1, n_e), jnp.float32),
            pltpu.VMEM((1, n_e), jnp.float32),
        ],
    )(x, gate_w)


def _dispatch(eid_flat, w_flat, rank_flat, cnt16, *, t_tok, n_e, npad, nt_pad):
    mesh = plsc.VectorSubcoreMesh(core_axis_name="c", subcore_axis_name="s")

    @functools.partial(
        pl.kernel, mesh=mesh,
        compiler_params=pltpu.CompilerParams(needs_layout_passes=False),
        out_type=[
            jax.ShapeDtypeStruct((npad,), jnp.int32),
            jax.ShapeDtypeStruct((npad,), jnp.float32),
            jax.ShapeDtypeStruct((2 * t_tok,), jnp.int32),
            jax.ShapeDtypeStruct((nt_pad,), jnp.int32),
        ],
        scratch_types=[
            pltpu.VMEM((2 * t_tok,), jnp.int32),
            pltpu.VMEM((2 * t_tok,), jnp.float32),
            pltpu.VMEM((npad,), jnp.float32),
            pltpu.VMEM((t_tok * n_e,), jnp.int32),
            pltpu.VMEM((16,), jnp.int32),
            pltpu.VMEM((npad,), jnp.int32),
            pltpu.VMEM((2 * t_tok,), jnp.int32),
            pltpu.VMEM((nt_pad,), jnp.int32),
            pltpu.VMEM((16,), jnp.int32),
        ],
    )
    def k2(eid_hbm, w_hbm, rank_hbm, cnt_hbm, tok_hbm, ws_hbm, pos_hbm, te_hbm,
           eid_v, w_v, ws_v, rank_v, cnt_v, tok_v, pos_v, te_v, off_v):
        cid = lax.axis_index("c")
        sid = lax.axis_index("s")

        @pl.when((cid == 0) & (sid == 0))
        def _work():
            pltpu.sync_copy(eid_hbm, eid_v)
            pltpu.sync_copy(w_hbm, w_v)
            pltpu.sync_copy(rank_hbm, rank_v)
            pltpu.sync_copy(cnt_hbm, cnt_v)
            lane = lax.iota(jnp.int32, 16)
            cv = jnp.where(lane < n_e, cnt_v[...], 0)
            pc = ((cv + BT - 1) // BT) * BT
            oi = plsc.cumsum(pc)
            off_v[...] = oi - pc

            for jc in range(nt_pad // 16):
                jv = (jc * 16 + lane) * BT
                acc = jnp.zeros((16,), jnp.int32)
                for e in range(n_e):
                    oe_bc = _lane_bcast(oi, e)
                    acc += (jv >= oe_bc).astype(jnp.int32)
                te_v[pl.ds(jc * 16, 16)] = jnp.minimum(acc, n_e - 1)
            pltpu.sync_copy(te_v, te_hbm)

            @plsc.parallel_loop(0, npad, step=16, unroll=8)
            def zinit(i):
                tok_v[pl.ds(i, 16)] = jnp.zeros((16,), jnp.int32)
                ws_v[pl.ds(i, 16)] = jnp.zeros((16,), jnp.float32)

            def body(c, carry):
                tvec = c * 16 + lane
                for kk in range(2):
                    ev = eid_v[pl.ds(kk * t_tok + c * 16, 16)]
                    wv = w_v[pl.ds(kk * t_tok + c * 16, 16)]
                    rv = plsc.load_gather(rank_v, [tvec * n_e + ev])
                    ov = plsc.load_gather(off_v, [ev])
                    pv = rv + ov
                    plsc.store_scatter(tok_v, [pv], tvec)
                    plsc.store_scatter(ws_v, [pv], wv)
                    pos_v[pl.ds(kk * t_tok + c * 16, 16)] = pv
                return carry
            lax.fori_loop(0, t_tok // 16, body, 0)
            pltpu.sync_copy(tok_v, tok_hbm)
            pltpu.sync_copy(ws_v, ws_hbm)
            pltpu.sync_copy(pos_v, pos_hbm)

    return k2(eid_flat, w_flat, rank_flat, cnt16)


def _gather_rows(x, tok_sorted, *, npad, h):
    mesh = plsc.VectorSubcoreMesh(core_axis_name="c", subcore_axis_name="s")
    rows_per = npad // 32
    chunk = 32
    n_chunks = rows_per // chunk

    @functools.partial(
        pl.kernel, mesh=mesh,
        compiler_params=pltpu.CompilerParams(needs_layout_passes=False),
        out_type=jax.ShapeDtypeStruct((npad, h), jnp.int32),
        scratch_types=[
            pltpu.VMEM((rows_per,), jnp.int32),
            pltpu.VMEM((chunk, h), jnp.int32),
            pltpu.VMEM((chunk, h), jnp.int32),
            pltpu.SemaphoreType.DMA,
            pltpu.SemaphoreType.DMA,
            pltpu.SemaphoreType.DMA,
            pltpu.SemaphoreType.DMA,
        ],
    )
    def k3(x_hbm, tok_hbm, out_hbm, idx_v, bufa, bufb, sga, sgb, ssa, ssb):
        wid = lax.axis_index("s") * 2 + lax.axis_index("c")
        base = wid * rows_per
        pltpu.sync_copy(tok_hbm.at[pl.ds(base, rows_per)], idx_v)
        buf = (bufa, bufb)
        sg = (sga, sgb)
        ss = (ssa, ssb)

        def issue(c):
            u = c % 2
            return pltpu.async_copy(
                x_hbm.at[idx_v.at[pl.ds(c * chunk, chunk)]], buf[u], sg[u])

        pend = {0: issue(0)}
        st = {}
        for c in range(n_chunks):
            u = c % 2
            if c + 1 < n_chunks:
                if c >= 1:
                    st[c - 1].wait()
                pend[c + 1] = issue(c + 1)
            pend.pop(c).wait()
            st[c] = pltpu.async_copy(
                buf[u], out_hbm.at[pl.ds(base + c * chunk, chunk)], ss[u])
        st[n_chunks - 2].wait()
        st[n_chunks - 1].wait()

    return k3(x, tok_sorted)


def _ffn_body(te_ref, xs_ref, wg_ref, wu_ref, wd_ref, ws_ref, y_ref):
    xp = xs_ref[...]
    hh = xp.shape[1]
    xa = lax.bitcast_convert_type((xp & 0xFFFF).astype(jnp.uint16),
                                  jnp.bfloat16)
    xb = lax.bitcast_convert_type(
        lax.shift_right_logical(xp, 16).astype(jnp.uint16), jnp.bfloat16)
    wg = wg_ref[0].astype(jnp.bfloat16)
    wu = wu_ref[0].astype(jnp.bfloat16)
    wd = wd_ref[0].astype(jnp.bfloat16)
    dn = (((1,), (0,)), ((), ()))
    g = (lax.dot_general(xa, wg[:hh], dn, preferred_element_type=jnp.float32)
         + lax.dot_general(xb, wg[hh:], dn, preferred_element_type=jnp.float32))
    u = (lax.dot_general(xa, wu[:hh], dn, preferred_element_type=jnp.float32)
         + lax.dot_general(xb, wu[hh:], dn, preferred_element_type=jnp.float32))
    act = ((g / (1.0 + jnp.exp(-g))) * u).astype(jnp.bfloat16)
    y = lax.dot_general(act, wd, dn, preferred_element_type=jnp.float32)
    bt = y.shape[0]
    ident = (lax.broadcasted_iota(jnp.int32, (bt, bt), 0)
             == lax.broadcasted_iota(jnp.int32, (bt, bt), 1)).astype(jnp.float32)
    wcol = lax.dot_general(ident, ws_ref[0], (((1,), (1,)), ((), ())),
                           preferred_element_type=jnp.float32)
    y_ref[...] = y * wcol


def _ffn(tile_eid, x_sorted, w_gate, w_up, w_down, w_sorted, *, npad, h, f):
    nt = npad // BT
    grid_spec = pltpu.PrefetchScalarGridSpec(
        num_scalar_prefetch=1,
        grid=(nt,),
        in_specs=[
            pl.BlockSpec((BT, h // 2), lambda j, te: (j, 0)),
            pl.BlockSpec((1, h, f), lambda j, te: (te[j], 0, 0)),
            pl.BlockSpec((1, h, f), lambda j, te: (te[j], 0, 0)),
            pl.BlockSpec((1, f, h), lambda j, te: (te[j], 0, 0)),
            pl.BlockSpec((1, 1, BT), lambda j, te: (j, 0, 0)),
        ],
        out_specs=pl.BlockSpec((BT, h), lambda j, te: (j, 0)),
    )
    return pl.pallas_call(
        _ffn_body,
        grid_spec=grid_spec,
        out_shape=jax.ShapeDtypeStruct((npad, h), jnp.float32),
    )(tile_eid, x_sorted, w_gate, w_up, w_down,
      w_sorted.reshape(nt, 1, BT))


def _combine(pos, y, *, t_tok, h):
    mesh = plsc.VectorSubcoreMesh(core_axis_name="c", subcore_axis_name="s")
    toks_per = t_tok // 32
    tc = 8
    n_chunks = toks_per // tc

    @functools.partial(
        pl.kernel, mesh=mesh,
        compiler_params=pltpu.CompilerParams(needs_layout_passes=False),
        out_type=jax.ShapeDtypeStruct((t_tok, h), jnp.float32),
        scratch_types=[
            pltpu.VMEM((toks_per,), jnp.int32),
            pltpu.VMEM((toks_per,), jnp.int32),
            pltpu.VMEM((tc, h), jnp.float32),
            pltpu.VMEM((tc, h), jnp.float32),
            pltpu.VMEM((tc, h), jnp.float32),
            pltpu.VMEM((tc, h), jnp.float32),
            pltpu.SemaphoreType.DMA,
            pltpu.SemaphoreType.DMA,
            pltpu.SemaphoreType.DMA,
            pltpu.SemaphoreType.DMA,
        ],
    )
    def k5(pos_hbm, y_hbm, out_hbm,
           p1_v, p2_v, b1a, b1b, b2a, b2b, sga, sgb, ssa, ssb):
        wid = lax.axis_index("s") * 2 + lax.axis_index("c")
        base = wid * toks_per
        pltpu.sync_copy(pos_hbm.at[pl.ds(base, toks_per)], p1_v)
        pltpu.sync_copy(pos_hbm.at[pl.ds(t_tok + base, toks_per)], p2_v)
        b1 = (b1a, b1b)
        b2 = (b2a, b2b)
        sg = (sga, sgb)
        ss = (ssa, ssb)

        def issue(c):
            u = c % 2
            cp1 = pltpu.async_copy(y_hbm.at[p1_v.at[pl.ds(c * tc, tc)]], b1[u], sg[u])
            cp2 = pltpu.async_copy(y_hbm.at[p2_v.at[pl.ds(c * tc, tc)]], b2[u], sg[u])
            return cp1, cp2

        pend = {0: issue(0)}
        st = {}
        for c in range(n_chunks):
            u = c % 2
            if c + 1 < n_chunks:
                if c >= 1:
                    st[c - 1].wait()
                pend[c + 1] = issue(c + 1)
            cp1, cp2 = pend.pop(c)
            cp1.wait()
            cp2.wait()
            for i in range(tc):
                @plsc.parallel_loop(0, h, step=16, unroll=8)
                def hh(hx, i=i, u=u):
                    sl = pl.ds(hx, 16)
                    b1[u][i, sl] += b2[u][i, sl]
            st[c] = pltpu.async_copy(b1[u], out_hbm.at[pl.ds(base + c * tc, tc)], ss[u])
        st[n_chunks - 2].wait()
        st[n_chunks - 1].wait()

    return k5(pos, y)


def kernel(hidden_states, gate_w, w_gate, w_up, w_down):
    b, s, h = hidden_states.shape
    n_e, _, f = w_gate.shape
    t_tok = b * s
    npad = 2 * t_tok + n_e * BT
    nt = npad // BT
    nt_pad = ((nt + 15) // 16) * 16
    x = hidden_states.reshape(t_tok, h)

    eid, w, rank, cnt, xp, aux = _router(x, gate_w)
    tok_sorted, w_sorted, pos, tile_eid = _dispatch(
        eid.reshape(-1), w.reshape(-1), rank.reshape(-1), cnt.reshape(-1),
        t_tok=t_tok, n_e=n_e, npad=npad, nt_pad=nt_pad)
    x_sorted_p = _gather_rows(xp, tok_sorted, npad=npad, h=h // 2)
    y = _ffn(tile_eid[:nt], x_sorted_p, w_gate, w_up, w_down, w_sorted,
             npad=npad, h=h, f=f)
    out = _combine(pos, y, t_tok=t_tok, h=h)
    return out.reshape(b, s, h), aux[0, 0]
